# baseline (speedup 1.0000x reference)
import functools

import jax
import jax.numpy as jnp
from jax.experimental import pallas as pl

EPS = 0.001
F_DIM = 128
N_RBF = 16
CUTOFF = 5.0
NUM_CONV = 2
N_NODES = 128

_F32 = jnp.float32


def _swish(x):
    return x * jax.nn.sigmoid(x)


def _dot(a, b):
    return jax.lax.dot_general(
        a.astype(jnp.bfloat16), b.astype(jnp.bfloat16),
        (((1,), (0,)), ((), ())), preferred_element_type=_F32)


def _fused_kernel(H_ref, adj_ref, adjT_ref, xyz_ref, xyzT_ref, *refs):
    F = F_DIM
    conv_refs = []
    idx = 0
    rest = refs
    for _ in range(NUM_CONV):
        conv_refs.append(rest[idx:idx + 11])
        idx += 11
    (Wmu1_r, bmu1_r, Wmu2_r, bmu2_r,
     Wsg1_r, bsg1_r, Wsg2_r, bsg2_r) = rest[idx:idx + 8]
    idx += 8
    mu_ref, sig_ref = rest[idx], rest[idx + 1]

    s = H_ref[0]
    adj = adj_ref[0]
    adjT = adjT_ref[0]
    xyz = xyz_ref[0]
    xyzT = xyzT_ref[0]

    deg_i = jnp.sum(adj, axis=1, keepdims=True)
    deg_j = jnp.sum(adjT, axis=0, keepdims=True)
    dis_i = jnp.sqrt(1.0 / deg_i + EPS)
    dis_j = jnp.sqrt(1.0 / deg_j + EPS)

    xi, yi, zi = xyz[:, 0:1], xyz[:, 1:2], xyz[:, 2:3]
    xj, yj, zj = xyzT[0:1, :], xyzT[1:2, :], xyzT[2:3, :]
    rx = xj - xi
    ry = yj - yi
    rz = zj - zi
    dist2 = rx * rx + ry * ry + rz * rz + 1e-9
    inv_dist = jax.lax.rsqrt(dist2)
    dist = dist2 * inv_dist
    ux = rx * inv_dist
    uy = ry * inv_dist
    uz = rz * inv_dist

    t = (jnp.pi / CUTOFF) * dist
    c1 = jnp.cos(t)
    s1 = jnp.sin(t)
    env = jnp.where(dist <= CUTOFF, 0.5 * (c1 + 1.0), 0.0)

    mask = (adj > 0.0).astype(_F32)
    eew = dis_i * dis_j * mask * env
    eewd = eew * inv_dist

    sins = [s1]
    for _ in range(N_RBF - 1):
        sins.append(2.0 * c1 * sins[-1] - (sins[-2] if len(sins) > 1 else jnp.zeros_like(s1)))
    G = [eewd * sk for sk in sins]
    G.append(eew)
    Gux = [Gk * ux for Gk in G]
    Guy = [Gk * uy for Gk in G]
    Guz = [Gk * uz for Gk in G]

    v0 = v1 = v2 = None
    for c in range(NUM_CONV):
        (Wm1_r, bm1_r, Wm2_r, bm2_r, C_r,
         U_r, V_r, Wu1_r, bu1_r, Wu2_r, bu2_r) = conv_refs[c]

        phi = _dot(_swish(_dot(s, Wm1_r[...]) + bm1_r[...]), Wm2_r[...]) + bm2_r[...]
        phi0 = phi[:, :F]
        phi1 = phi[:, F:2 * F]
        phi2 = phi[:, 2 * F:]
        C = C_r[...]

        ds = jnp.zeros((N_NODES, F), _F32)
        dA0 = jnp.zeros((N_NODES, F), _F32)
        dA1 = jnp.zeros((N_NODES, F), _F32)
        dA2 = jnp.zeros((N_NODES, F), _F32)
        if c == 0:
            for k in range(N_RBF + 1):
                ds = ds + _dot(G[k], phi1) * C[k:k + 1, F:2 * F]
                p2k = phi2 * C[k:k + 1, 2 * F:]
                dA0 = dA0 + _dot(Gux[k], p2k)
                dA1 = dA1 + _dot(Guy[k], p2k)
                dA2 = dA2 + _dot(Guz[k], p2k)
            s = s + ds
            v0, v1, v2 = dA0, dA1, dA2
        else:
            R = jnp.concatenate([phi1, phi0 * v0, phi0 * v1, phi0 * v2], axis=1)
            dB0 = jnp.zeros((N_NODES, F), _F32)
            dB1 = jnp.zeros((N_NODES, F), _F32)
            dB2 = jnp.zeros((N_NODES, F), _F32)
            for k in range(N_RBF + 1):
                M = _dot(G[k], R)
                ck0 = C[k:k + 1, :F]
                ds = ds + M[:, :F] * C[k:k + 1, F:2 * F]
                dB0 = dB0 + M[:, F:2 * F] * ck0
                dB1 = dB1 + M[:, 2 * F:3 * F] * ck0
                dB2 = dB2 + M[:, 3 * F:] * ck0
                p2k = phi2 * C[k:k + 1, 2 * F:]
                dA0 = dA0 + _dot(Gux[k], p2k)
                dA1 = dA1 + _dot(Guy[k], p2k)
                dA2 = dA2 + _dot(Guz[k], p2k)
            s = s + ds
            v0 = v0 + dA0 + dB0
            v1 = v1 + dA1 + dB1
            v2 = v2 + dA2 + dB2

        U = U_r[...]
        V = V_r[...]
        uv0, uv1, uv2 = _dot(v0, U), _dot(v1, U), _dot(v2, U)
        vv0, vv1, vv2 = _dot(v0, V), _dot(v1, V), _dot(v2, V)
        vnorm = jnp.sqrt(vv0 * vv0 + vv1 * vv1 + vv2 * vv2 + 1e-8)
        stack = jnp.concatenate([s, vnorm], axis=1)
        inner = _swish(_dot(stack, Wu1_r[...]) + bu1_r[...])
        split = _dot(inner, Wu2_r[...]) + bu2_r[...]
        a_vv = split[:, :F]
        a_sv = split[:, F:2 * F]
        a_ss = split[:, 2 * F:]
        s = s + a_sv * (uv0 * vv0 + uv1 * vv1 + uv2 * vv2) + a_ss
        v0 = v0 + uv0 * a_vv
        v1 = v1 + uv1 * a_vv
        v2 = v2 + uv2 * a_vv

    mu_ref[0] = _dot(jnp.tanh(_dot(s, Wmu1_r[...]) + bmu1_r[...]), Wmu2_r[...]) + bmu2_r[...]
    logvar = _dot(jnp.tanh(_dot(s, Wsg1_r[...]) + bsg1_r[...]), Wsg2_r[...]) + bsg2_r[...]
    sig_ref[0] = 1e-9 + jnp.exp(logvar * 0.5)


@jax.jit
def kernel(H, cg_adj, cg_xyz, params):
    B, N, F = H.shape
    bf16 = jnp.bfloat16
    w_args = []
    for p in params['convs']:
        w_args += [
            p['Wm1'].astype(bf16), p['bm1'].reshape(1, -1),
            p['Wm2'].astype(bf16), p['bm2'].reshape(1, -1),
            jnp.concatenate([p['Wd'], p['bd'][None, :]], axis=0),
            p['U'].astype(bf16), p['V'].astype(bf16),
            p['Wu1'].astype(bf16), p['bu1'].reshape(1, -1),
            p['Wu2'].astype(bf16), p['bu2'].reshape(1, -1),
        ]
    pm, ps = params['mu'], params['sigma']
    w_args += [
        pm['W1'].astype(bf16), pm['b1'].reshape(1, -1),
        pm['W2'].astype(bf16), pm['b2'].reshape(1, -1),
        ps['W1'].astype(bf16), ps['b1'].reshape(1, -1),
        ps['W2'].astype(bf16), ps['b2'].reshape(1, -1),
    ]

    adjT = jnp.swapaxes(cg_adj, 1, 2)
    xyzT = jnp.swapaxes(cg_xyz, 1, 2)

    def b_spec(shape):
        return pl.BlockSpec(shape, lambda b: (b,) + (0,) * (len(shape) - 1))

    def w_spec(a):
        nd = a.ndim
        return pl.BlockSpec(a.shape, lambda b: (0,) * nd)

    in_specs = [
        b_spec((1, N, F)),
        b_spec((1, N, N)),
        b_spec((1, N, N)),
        b_spec((1, N, 3)),
        b_spec((1, 3, N)),
    ] + [w_spec(a) for a in w_args]

    out_shape = [
        jax.ShapeDtypeStruct((B, N, F), H.dtype),
        jax.ShapeDtypeStruct((B, N, F), H.dtype),
    ]
    out_specs = [b_spec((1, N, F)), b_spec((1, N, F))]

    H_mu, H_sigma = pl.pallas_call(
        _fused_kernel,
        grid=(B,),
        in_specs=in_specs,
        out_specs=out_specs,
        out_shape=out_shape,
    )(H, cg_adj, adjT, cg_xyz, xyzT, *w_args)
    return H_mu, H_sigma


# speedup vs baseline: 1.2854x; 1.2854x over previous
import functools

import jax
import jax.numpy as jnp
from jax.experimental import pallas as pl

EPS = 0.001
F_DIM = 128
N_RBF = 16
CUTOFF = 5.0
NUM_CONV = 2
N_NODES = 128

_F32 = jnp.float32


def _swish(x):
    return x * jax.nn.sigmoid(x)


def _dot(a, b):
    return jax.lax.dot_general(
        a, b, (((1,), (0,)), ((), ())), preferred_element_type=_F32)


def _fused_kernel(H_ref, adj_ref, adjT_ref, xyz_ref, xyzT_ref, *refs):
    F = F_DIM
    conv_refs = []
    idx = 0
    rest = refs
    for _ in range(NUM_CONV):
        conv_refs.append(rest[idx:idx + 11])
        idx += 11
    (Wmu1_r, bmu1_r, Wmu2_r, bmu2_r,
     Wsg1_r, bsg1_r, Wsg2_r, bsg2_r) = rest[idx:idx + 8]
    idx += 8
    mu_ref, sig_ref = rest[idx], rest[idx + 1]

    s = H_ref[0]
    adj = adj_ref[0]
    adjT = adjT_ref[0]
    xyz = xyz_ref[0]
    xyzT = xyzT_ref[0]

    deg_i = jnp.sum(adj, axis=1, keepdims=True)
    deg_j = jnp.sum(adjT, axis=0, keepdims=True)
    dis_i = jnp.sqrt(1.0 / deg_i + EPS)
    dis_j = jnp.sqrt(1.0 / deg_j + EPS)

    xi, yi, zi = xyz[:, 0:1], xyz[:, 1:2], xyz[:, 2:3]
    xj, yj, zj = xyzT[0:1, :], xyzT[1:2, :], xyzT[2:3, :]
    rx = xj - xi
    ry = yj - yi
    rz = zj - zi
    dist2 = rx * rx + ry * ry + rz * rz + 1e-9
    inv_dist = jax.lax.rsqrt(dist2)
    dist = dist2 * inv_dist
    ux = rx * inv_dist
    uy = ry * inv_dist
    uz = rz * inv_dist

    t = (jnp.pi / CUTOFF) * dist
    c1 = jnp.cos(t)
    s1 = jnp.sin(t)
    env = jnp.where(dist <= CUTOFF, 0.5 * (c1 + 1.0), 0.0)

    mask = (adj > 0.0).astype(_F32)
    eew = dis_i * dis_j * mask * env
    eewd = eew * inv_dist

    sins = [s1]
    for _ in range(N_RBF - 1):
        sins.append(2.0 * c1 * sins[-1] - (sins[-2] if len(sins) > 1 else jnp.zeros_like(s1)))
    G = [eewd * sk for sk in sins]
    G.append(eew)
    Gux = [Gk * ux for Gk in G]
    Guy = [Gk * uy for Gk in G]
    Guz = [Gk * uz for Gk in G]

    v0 = v1 = v2 = None
    for c in range(NUM_CONV):
        (Wm1_r, bm1_r, Wm2_r, bm2_r, C_r,
         U_r, V_r, Wu1_r, bu1_r, Wu2_r, bu2_r) = conv_refs[c]

        phi = _dot(_swish(_dot(s, Wm1_r[...]) + bm1_r[...]), Wm2_r[...]) + bm2_r[...]
        phi0 = phi[:, :F]
        phi1 = phi[:, F:2 * F]
        phi2 = phi[:, 2 * F:]
        C = C_r[...]

        ds = jnp.zeros((N_NODES, F), _F32)
        dA0 = jnp.zeros((N_NODES, F), _F32)
        dA1 = jnp.zeros((N_NODES, F), _F32)
        dA2 = jnp.zeros((N_NODES, F), _F32)
        if c == 0:
            for k in range(N_RBF + 1):
                ds = ds + _dot(G[k], phi1) * C[k:k + 1, F:2 * F]
                p2k = phi2 * C[k:k + 1, 2 * F:]
                dA0 = dA0 + _dot(Gux[k], p2k)
                dA1 = dA1 + _dot(Guy[k], p2k)
                dA2 = dA2 + _dot(Guz[k], p2k)
            s = s + ds
            v0, v1, v2 = dA0, dA1, dA2
        else:
            R = jnp.concatenate([phi1, phi0 * v0, phi0 * v1, phi0 * v2], axis=1)
            dB0 = jnp.zeros((N_NODES, F), _F32)
            dB1 = jnp.zeros((N_NODES, F), _F32)
            dB2 = jnp.zeros((N_NODES, F), _F32)
            for k in range(N_RBF + 1):
                M = _dot(G[k], R)
                ck0 = C[k:k + 1, :F]
                ds = ds + M[:, :F] * C[k:k + 1, F:2 * F]
                dB0 = dB0 + M[:, F:2 * F] * ck0
                dB1 = dB1 + M[:, 2 * F:3 * F] * ck0
                dB2 = dB2 + M[:, 3 * F:] * ck0
                p2k = phi2 * C[k:k + 1, 2 * F:]
                dA0 = dA0 + _dot(Gux[k], p2k)
                dA1 = dA1 + _dot(Guy[k], p2k)
                dA2 = dA2 + _dot(Guz[k], p2k)
            s = s + ds
            v0 = v0 + dA0 + dB0
            v1 = v1 + dA1 + dB1
            v2 = v2 + dA2 + dB2

        U = U_r[...]
        V = V_r[...]
        uv0, uv1, uv2 = _dot(v0, U), _dot(v1, U), _dot(v2, U)
        vv0, vv1, vv2 = _dot(v0, V), _dot(v1, V), _dot(v2, V)
        vnorm = jnp.sqrt(vv0 * vv0 + vv1 * vv1 + vv2 * vv2 + 1e-8)
        stack = jnp.concatenate([s, vnorm], axis=1)
        inner = _swish(_dot(stack, Wu1_r[...]) + bu1_r[...])
        split = _dot(inner, Wu2_r[...]) + bu2_r[...]
        a_vv = split[:, :F]
        a_sv = split[:, F:2 * F]
        a_ss = split[:, 2 * F:]
        s = s + a_sv * (uv0 * vv0 + uv1 * vv1 + uv2 * vv2) + a_ss
        v0 = v0 + uv0 * a_vv
        v1 = v1 + uv1 * a_vv
        v2 = v2 + uv2 * a_vv

    mu_ref[0] = _dot(jnp.tanh(_dot(s, Wmu1_r[...]) + bmu1_r[...]), Wmu2_r[...]) + bmu2_r[...]
    logvar = _dot(jnp.tanh(_dot(s, Wsg1_r[...]) + bsg1_r[...]), Wsg2_r[...]) + bsg2_r[...]
    sig_ref[0] = 1e-9 + jnp.exp(logvar * 0.5)


@jax.jit
def kernel(H, cg_adj, cg_xyz, params):
    B, N, F = H.shape
    w_args = []
    for p in params['convs']:
        w_args += [
            p['Wm1'], p['bm1'].reshape(1, -1),
            p['Wm2'], p['bm2'].reshape(1, -1),
            jnp.concatenate([p['Wd'], p['bd'][None, :]], axis=0),
            p['U'], p['V'],
            p['Wu1'], p['bu1'].reshape(1, -1),
            p['Wu2'], p['bu2'].reshape(1, -1),
        ]
    pm, ps = params['mu'], params['sigma']
    w_args += [
        pm['W1'], pm['b1'].reshape(1, -1),
        pm['W2'], pm['b2'].reshape(1, -1),
        ps['W1'], ps['b1'].reshape(1, -1),
        ps['W2'], ps['b2'].reshape(1, -1),
    ]

    adjT = jnp.swapaxes(cg_adj, 1, 2)
    xyzT = jnp.swapaxes(cg_xyz, 1, 2)

    def b_spec(shape):
        return pl.BlockSpec(shape, lambda b: (b,) + (0,) * (len(shape) - 1))

    def w_spec(a):
        nd = a.ndim
        return pl.BlockSpec(a.shape, lambda b: (0,) * nd)

    in_specs = [
        b_spec((1, N, F)),
        b_spec((1, N, N)),
        b_spec((1, N, N)),
        b_spec((1, N, 3)),
        b_spec((1, 3, N)),
    ] + [w_spec(a) for a in w_args]

    out_shape = [
        jax.ShapeDtypeStruct((B, N, F), H.dtype),
        jax.ShapeDtypeStruct((B, N, F), H.dtype),
    ]
    out_specs = [b_spec((1, N, F)), b_spec((1, N, F))]

    H_mu, H_sigma = pl.pallas_call(
        _fused_kernel,
        grid=(B,),
        in_specs=in_specs,
        out_specs=out_specs,
        out_shape=out_shape,
    )(H, cg_adj, adjT, cg_xyz, xyzT, *w_args)
    return H_mu, H_sigma


# speedup vs baseline: 1.4394x; 1.1198x over previous
import jax
import jax.numpy as jnp
from jax.experimental import pallas as pl

EPS = 0.001
F_DIM = 128
N_RBF = 16
CUTOFF = 5.0
NUM_CONV = 2
B_SZ = 4
N_NODES = 128

_F32 = jnp.float32


def _swish(x):
    return x * jax.nn.sigmoid(x)


def _dot(a, b):
    return jax.lax.dot_general(
        a, b, (((1,), (0,)), ((), ())), preferred_element_type=_F32)


def _fused_kernel(H_ref, adj_ref, xyz_ref, *refs):
    F = F_DIM
    conv_refs = []
    idx = 0
    for _ in range(NUM_CONV):
        conv_refs.append(refs[idx:idx + 11])
        idx += 11
    (Wmu1_r, bmu1_r, Wmu2_r, bmu2_r,
     Wsg1_r, bsg1_r, Wsg2_r, bsg2_r) = refs[idx:idx + 8]
    idx += 8
    mu_ref, sig_ref = refs[idx], refs[idx + 1]

    for b in range(B_SZ):
        s = H_ref[b]
        adj = adj_ref[b]
        xyz = xyz_ref[b]

        deg_i = jnp.sum(adj, axis=1, keepdims=True)
        dis_i = jnp.sqrt(1.0 / deg_i + EPS)
        dis_j = jnp.transpose(dis_i)

        xi, yi, zi = xyz[:, 0:1], xyz[:, 1:2], xyz[:, 2:3]
        xj = jnp.transpose(xi)
        yj = jnp.transpose(yi)
        zj = jnp.transpose(zi)
        rx = xj - xi
        ry = yj - yi
        rz = zj - zi
        dist2 = rx * rx + ry * ry + rz * rz + 1e-9
        inv_dist = jax.lax.rsqrt(dist2)
        dist = dist2 * inv_dist
        ux = rx * inv_dist
        uy = ry * inv_dist
        uz = rz * inv_dist

        t = (jnp.pi / CUTOFF) * dist
        c1 = jnp.cos(t)
        s1 = jnp.sin(t)
        env = jnp.where(dist <= CUTOFF, 0.5 * (c1 + 1.0), 0.0)

        mask = (adj > 0.0).astype(_F32)
        eew = dis_i * dis_j * mask * env
        eewd = eew * inv_dist

        sins = [s1]
        for _ in range(N_RBF - 1):
            prev2 = sins[-2] if len(sins) > 1 else jnp.zeros_like(s1)
            sins.append(2.0 * c1 * sins[-1] - prev2)
        G = [eewd * sk for sk in sins]
        G.append(eew)
        Gux = [Gk * ux for Gk in G]
        Guy = [Gk * uy for Gk in G]
        Guz = [Gk * uz for Gk in G]

        v0 = v1 = v2 = None
        for c in range(NUM_CONV):
            (Wm1_r, bm1_r, Wm2_r, bm2_r, C_r,
             U_r, V_r, Wu1_r, bu1_r, Wu2_r, bu2_r) = conv_refs[c]

            phi = _dot(_swish(_dot(s, Wm1_r[...]) + bm1_r[...]), Wm2_r[...]) + bm2_r[...]
            phi0 = phi[:, :F]
            phi1 = phi[:, F:2 * F]
            phi2 = phi[:, 2 * F:]
            C = C_r[...]

            ds = jnp.zeros((N_NODES, F), _F32)
            dA0 = jnp.zeros((N_NODES, F), _F32)
            dA1 = jnp.zeros((N_NODES, F), _F32)
            dA2 = jnp.zeros((N_NODES, F), _F32)
            if c == 0:
                for k in range(N_RBF + 1):
                    ds = ds + _dot(G[k], phi1) * C[k:k + 1, F:2 * F]
                    p2k = phi2 * C[k:k + 1, 2 * F:]
                    dA0 = dA0 + _dot(Gux[k], p2k)
                    dA1 = dA1 + _dot(Guy[k], p2k)
                    dA2 = dA2 + _dot(Guz[k], p2k)
                s = s + ds
                v0, v1, v2 = dA0, dA1, dA2
            else:
                R = jnp.concatenate([phi1, phi0 * v0, phi0 * v1, phi0 * v2], axis=1)
                dB0 = jnp.zeros((N_NODES, F), _F32)
                dB1 = jnp.zeros((N_NODES, F), _F32)
                dB2 = jnp.zeros((N_NODES, F), _F32)
                for k in range(N_RBF + 1):
                    M = _dot(G[k], R)
                    ck0 = C[k:k + 1, :F]
                    ds = ds + M[:, :F] * C[k:k + 1, F:2 * F]
                    dB0 = dB0 + M[:, F:2 * F] * ck0
                    dB1 = dB1 + M[:, 2 * F:3 * F] * ck0
                    dB2 = dB2 + M[:, 3 * F:] * ck0
                    p2k = phi2 * C[k:k + 1, 2 * F:]
                    dA0 = dA0 + _dot(Gux[k], p2k)
                    dA1 = dA1 + _dot(Guy[k], p2k)
                    dA2 = dA2 + _dot(Guz[k], p2k)
                s = s + ds
                v0 = v0 + dA0 + dB0
                v1 = v1 + dA1 + dB1
                v2 = v2 + dA2 + dB2

            U = U_r[...]
            V = V_r[...]
            uv0, uv1, uv2 = _dot(v0, U), _dot(v1, U), _dot(v2, U)
            vv0, vv1, vv2 = _dot(v0, V), _dot(v1, V), _dot(v2, V)
            vnorm = jnp.sqrt(vv0 * vv0 + vv1 * vv1 + vv2 * vv2 + 1e-8)
            stack = jnp.concatenate([s, vnorm], axis=1)
            inner = _swish(_dot(stack, Wu1_r[...]) + bu1_r[...])
            split = _dot(inner, Wu2_r[...]) + bu2_r[...]
            a_vv = split[:, :F]
            a_sv = split[:, F:2 * F]
            a_ss = split[:, 2 * F:]
            s = s + a_sv * (uv0 * vv0 + uv1 * vv1 + uv2 * vv2) + a_ss
            v0 = v0 + uv0 * a_vv
            v1 = v1 + uv1 * a_vv
            v2 = v2 + uv2 * a_vv

        mu_ref[b] = _dot(jnp.tanh(_dot(s, Wmu1_r[...]) + bmu1_r[...]), Wmu2_r[...]) + bmu2_r[...]
        logvar = _dot(jnp.tanh(_dot(s, Wsg1_r[...]) + bsg1_r[...]), Wsg2_r[...]) + bsg2_r[...]
        sig_ref[b] = 1e-9 + jnp.exp(logvar * 0.5)


@jax.jit
def kernel(H, cg_adj, cg_xyz, params):
    B, N, F = H.shape
    w_args = []
    for p in params['convs']:
        w_args += [
            p['Wm1'], p['bm1'].reshape(1, -1), p['Wm2'], p['bm2'].reshape(1, -1),
            jnp.concatenate([p['Wd'], p['bd'][None, :]], axis=0),
            p['U'], p['V'],
            p['Wu1'], p['bu1'].reshape(1, -1), p['Wu2'], p['bu2'].reshape(1, -1),
        ]
    pm, ps = params['mu'], params['sigma']
    w_args += [
        pm['W1'], pm['b1'].reshape(1, -1), pm['W2'], pm['b2'].reshape(1, -1),
        ps['W1'], ps['b1'].reshape(1, -1), ps['W2'], ps['b2'].reshape(1, -1),
    ]

    out_shape = [
        jax.ShapeDtypeStruct((B, N, F), H.dtype),
        jax.ShapeDtypeStruct((B, N, F), H.dtype),
    ]

    H_mu, H_sigma = pl.pallas_call(
        _fused_kernel,
        out_shape=out_shape,
    )(H, cg_adj, cg_xyz, *w_args)
    return H_mu, H_sigma


# speedup vs baseline: 1.8614x; 1.2932x over previous
import jax
import jax.numpy as jnp
from jax.experimental import pallas as pl

EPS = 0.001
F_DIM = 128
N_RBF = 16
CUTOFF = 5.0
NUM_CONV = 2
B_SZ = 4
N_NODES = 128

_F32 = jnp.float32


def _swish(x):
    return x * jax.nn.sigmoid(x)


def _dot(a, b):
    return jax.lax.dot_general(
        a, b, (((1,), (0,)), ((), ())), preferred_element_type=_F32)


def _fused_kernel(H_ref, adj_ref, xyz_ref, *refs):
    F = F_DIM
    conv_refs = []
    idx = 0
    for _ in range(NUM_CONV):
        conv_refs.append(refs[idx:idx + 12])
        idx += 12
    (Wmu1_r, bmu1_r, Wmu2_r, bmu2_r,
     Wsg1_r, bsg1_r, Wsg2_r, bsg2_r) = refs[idx:idx + 8]
    idx += 8
    mu_ref, sig_ref = refs[idx], refs[idx + 1]

    for b in range(B_SZ):
        s = H_ref[b]
        adj = adj_ref[b]
        xyz = xyz_ref[b]

        deg_i = jnp.sum(adj, axis=1, keepdims=True)
        dis_i = jnp.sqrt(1.0 / deg_i + EPS)
        dis_j = jnp.transpose(dis_i)

        xi, yi, zi = xyz[:, 0:1], xyz[:, 1:2], xyz[:, 2:3]
        xj = jnp.transpose(xi)
        yj = jnp.transpose(yi)
        zj = jnp.transpose(zi)
        rx = xj - xi
        ry = yj - yi
        rz = zj - zi
        dist2 = rx * rx + ry * ry + rz * rz + 1e-9
        inv_dist = jax.lax.rsqrt(dist2)
        dist = dist2 * inv_dist
        ux = rx * inv_dist
        uy = ry * inv_dist
        uz = rz * inv_dist

        t = (jnp.pi / CUTOFF) * dist
        c1 = jnp.cos(t)
        s1 = jnp.sin(t)
        env = jnp.where(dist <= CUTOFF, 0.5 * (c1 + 1.0), 0.0)

        mask = (adj > 0.0).astype(_F32)
        eew = dis_i * dis_j * mask * env
        eewd = eew * inv_dist

        sins = [s1]
        for _ in range(N_RBF - 1):
            prev2 = sins[-2] if len(sins) > 1 else jnp.zeros_like(s1)
            sins.append(2.0 * c1 * sins[-1] - prev2)
        G = [eewd * sk for sk in sins]
        G.append(eew)
        Gux = [Gk * ux for Gk in G]
        Guy = [Gk * uy for Gk in G]
        Guz = [Gk * uz for Gk in G]

        v0 = v1 = v2 = None
        for c in range(NUM_CONV):
            (Wm1_r, bm1_r, Wm2_r, bm2_r, Wd_r, bd_r,
             U_r, V_r, Wu1_r, bu1_r, Wu2_r, bu2_r) = conv_refs[c]

            bm1 = jnp.reshape(bm1_r[...], (1, F))
            bm2 = jnp.reshape(bm2_r[...], (1, 3 * F))
            phi = _dot(_swish(_dot(s, Wm1_r[...]) + bm1), Wm2_r[...]) + bm2
            phi0 = phi[:, :F]
            phi1 = phi[:, F:2 * F]
            phi2 = phi[:, 2 * F:]
            C = jnp.concatenate(
                [Wd_r[...], jnp.reshape(bd_r[...], (1, 3 * F))], axis=0)

            ds = jnp.zeros((N_NODES, F), _F32)
            dA0 = jnp.zeros((N_NODES, F), _F32)
            dA1 = jnp.zeros((N_NODES, F), _F32)
            dA2 = jnp.zeros((N_NODES, F), _F32)
            if c == 0:
                for k in range(N_RBF + 1):
                    ds = ds + _dot(G[k], phi1) * C[k:k + 1, F:2 * F]
                    p2k = phi2 * C[k:k + 1, 2 * F:]
                    dA0 = dA0 + _dot(Gux[k], p2k)
                    dA1 = dA1 + _dot(Guy[k], p2k)
                    dA2 = dA2 + _dot(Guz[k], p2k)
                s = s + ds
                v0, v1, v2 = dA0, dA1, dA2
            else:
                R = jnp.concatenate([phi1, phi0 * v0, phi0 * v1, phi0 * v2], axis=1)
                dB0 = jnp.zeros((N_NODES, F), _F32)
                dB1 = jnp.zeros((N_NODES, F), _F32)
                dB2 = jnp.zeros((N_NODES, F), _F32)
                for k in range(N_RBF + 1):
                    M = _dot(G[k], R)
                    ck0 = C[k:k + 1, :F]
                    ds = ds + M[:, :F] * C[k:k + 1, F:2 * F]
                    dB0 = dB0 + M[:, F:2 * F] * ck0
                    dB1 = dB1 + M[:, 2 * F:3 * F] * ck0
                    dB2 = dB2 + M[:, 3 * F:] * ck0
                    p2k = phi2 * C[k:k + 1, 2 * F:]
                    dA0 = dA0 + _dot(Gux[k], p2k)
                    dA1 = dA1 + _dot(Guy[k], p2k)
                    dA2 = dA2 + _dot(Guz[k], p2k)
                s = s + ds
                v0 = v0 + dA0 + dB0
                v1 = v1 + dA1 + dB1
                v2 = v2 + dA2 + dB2

            U = U_r[...]
            V = V_r[...]
            uv0, uv1, uv2 = _dot(v0, U), _dot(v1, U), _dot(v2, U)
            vv0, vv1, vv2 = _dot(v0, V), _dot(v1, V), _dot(v2, V)
            vnorm = jnp.sqrt(vv0 * vv0 + vv1 * vv1 + vv2 * vv2 + 1e-8)
            stack = jnp.concatenate([s, vnorm], axis=1)
            inner = _swish(_dot(stack, Wu1_r[...]) + jnp.reshape(bu1_r[...], (1, F)))
            split = _dot(inner, Wu2_r[...]) + jnp.reshape(bu2_r[...], (1, 3 * F))
            a_vv = split[:, :F]
            a_sv = split[:, F:2 * F]
            a_ss = split[:, 2 * F:]
            s = s + a_sv * (uv0 * vv0 + uv1 * vv1 + uv2 * vv2) + a_ss
            v0 = v0 + uv0 * a_vv
            v1 = v1 + uv1 * a_vv
            v2 = v2 + uv2 * a_vv

        mu_ref[b] = (_dot(jnp.tanh(_dot(s, Wmu1_r[...]) + jnp.reshape(bmu1_r[...], (1, F))), Wmu2_r[...])
                     + jnp.reshape(bmu2_r[...], (1, F)))
        logvar = (_dot(jnp.tanh(_dot(s, Wsg1_r[...]) + jnp.reshape(bsg1_r[...], (1, F))), Wsg2_r[...])
                  + jnp.reshape(bsg2_r[...], (1, F)))
        sig_ref[b] = 1e-9 + jnp.exp(logvar * 0.5)


@jax.jit
def kernel(H, cg_adj, cg_xyz, params):
    B, N, F = H.shape
    w_args = []
    for p in params['convs']:
        w_args += [
            p['Wm1'], p['bm1'], p['Wm2'], p['bm2'], p['Wd'], p['bd'],
            p['U'], p['V'], p['Wu1'], p['bu1'], p['Wu2'], p['bu2'],
        ]
    pm, ps = params['mu'], params['sigma']
    w_args += [
        pm['W1'], pm['b1'], pm['W2'], pm['b2'],
        ps['W1'], ps['b1'], ps['W2'], ps['b2'],
    ]

    out_shape = [
        jax.ShapeDtypeStruct((B, N, F), H.dtype),
        jax.ShapeDtypeStruct((B, N, F), H.dtype),
    ]

    H_mu, H_sigma = pl.pallas_call(
        _fused_kernel,
        out_shape=out_shape,
    )(H, cg_adj, cg_xyz, *w_args)
    return H_mu, H_sigma


# speedup vs baseline: 2.0025x; 1.0758x over previous
import jax
import jax.numpy as jnp
from jax.experimental import pallas as pl

EPS = 0.001
F_DIM = 128
N_RBF = 16
CUTOFF = 5.0
NUM_CONV = 2
B_SZ = 4
N_NODES = 128

_F32 = jnp.float32


def _swish(x):
    return x * jax.nn.sigmoid(x)


def _dot(a, b):
    return jax.lax.dot_general(
        a, b, (((1,), (0,)), ((), ())), preferred_element_type=_F32)


def _fused_kernel(H_ref, adj_ref, xyz_ref, *refs):
    F = F_DIM
    conv_refs = []
    idx = 0
    for _ in range(NUM_CONV):
        conv_refs.append(refs[idx:idx + 12])
        idx += 12
    (Wmu1_r, bmu1_r, Wmu2_r, bmu2_r,
     Wsg1_r, bsg1_r, Wsg2_r, bsg2_r) = refs[idx:idx + 8]
    idx += 8
    mu_ref, sig_ref = refs[idx], refs[idx + 1]

    for b in range(B_SZ):
        s = H_ref[b]
        adj = adj_ref[b]
        xyz = xyz_ref[b]

        deg_i = jnp.sum(adj, axis=1, keepdims=True)
        dis_i = jnp.sqrt(1.0 / deg_i + EPS)
        dis_j = jnp.transpose(dis_i)

        xi, yi, zi = xyz[:, 0:1], xyz[:, 1:2], xyz[:, 2:3]
        xj = jnp.transpose(xi)
        yj = jnp.transpose(yi)
        zj = jnp.transpose(zi)
        rx = xj - xi
        ry = yj - yi
        rz = zj - zi
        dist2 = rx * rx + ry * ry + rz * rz + 1e-9
        inv_dist = jax.lax.rsqrt(dist2)
        dist = dist2 * inv_dist
        ux = rx * inv_dist
        uy = ry * inv_dist
        uz = rz * inv_dist

        t = (jnp.pi / CUTOFF) * dist
        c1 = jnp.cos(t)
        s1 = jnp.sin(t)
        env = jnp.where(dist <= CUTOFF, 0.5 * (c1 + 1.0), 0.0)

        mask = (adj > 0.0).astype(_F32)
        eew = dis_i * dis_j * mask * env
        eewd = eew * inv_dist

        sins = [s1]
        for _ in range(N_RBF - 1):
            prev2 = sins[-2] if len(sins) > 1 else jnp.zeros_like(s1)
            sins.append(2.0 * c1 * sins[-1] - prev2)
        G = [eewd * sk for sk in sins]
        G.append(eew)
        Ghat = jnp.concatenate(G, axis=1)
        Guxh = jnp.concatenate([g * ux for g in G], axis=1)
        Guyh = jnp.concatenate([g * uy for g in G], axis=1)
        Guzh = jnp.concatenate([g * uz for g in G], axis=1)

        v0 = v1 = v2 = None
        for c in range(NUM_CONV):
            (Wm1_r, bm1_r, Wm2_r, bm2_r, Wd_r, bd_r,
             U_r, V_r, Wu1_r, bu1_r, Wu2_r, bu2_r) = conv_refs[c]

            bm1 = jnp.reshape(bm1_r[...], (1, F))
            bm2 = jnp.reshape(bm2_r[...], (1, 3 * F))
            phi = _dot(_swish(_dot(s, Wm1_r[...]) + bm1), Wm2_r[...]) + bm2
            phi0 = phi[:, :F]
            phi1 = phi[:, F:2 * F]
            phi2 = phi[:, 2 * F:]
            C = jnp.concatenate(
                [Wd_r[...], jnp.reshape(bd_r[...], (1, 3 * F))], axis=0)

            Phi2 = jnp.concatenate(
                [phi2 * C[k:k + 1, 2 * F:] for k in range(N_RBF + 1)], axis=0)
            if c == 0:
                Phi1 = jnp.concatenate(
                    [phi1 * C[k:k + 1, F:2 * F] for k in range(N_RBF + 1)], axis=0)
                s = s + _dot(Ghat, Phi1)
                v0 = _dot(Guxh, Phi2)
                v1 = _dot(Guyh, Phi2)
                v2 = _dot(Guzh, Phi2)
            else:
                R = jnp.concatenate([phi1, phi0 * v0, phi0 * v1, phi0 * v2], axis=1)
                Crep = jnp.concatenate(
                    [C[:, F:2 * F], C[:, :F], C[:, :F], C[:, :F]], axis=1)
                PhiR = jnp.concatenate(
                    [R * Crep[k:k + 1, :] for k in range(N_RBF + 1)], axis=0)
                M = _dot(Ghat, PhiR)
                s = s + M[:, :F]
                v0 = v0 + M[:, F:2 * F] + _dot(Guxh, Phi2)
                v1 = v1 + M[:, 2 * F:3 * F] + _dot(Guyh, Phi2)
                v2 = v2 + M[:, 3 * F:] + _dot(Guzh, Phi2)

            U = U_r[...]
            V = V_r[...]
            uv0, uv1, uv2 = _dot(v0, U), _dot(v1, U), _dot(v2, U)
            vv0, vv1, vv2 = _dot(v0, V), _dot(v1, V), _dot(v2, V)
            vnorm = jnp.sqrt(vv0 * vv0 + vv1 * vv1 + vv2 * vv2 + 1e-8)
            stack = jnp.concatenate([s, vnorm], axis=1)
            inner = _swish(_dot(stack, Wu1_r[...]) + jnp.reshape(bu1_r[...], (1, F)))
            split = _dot(inner, Wu2_r[...]) + jnp.reshape(bu2_r[...], (1, 3 * F))
            a_vv = split[:, :F]
            a_sv = split[:, F:2 * F]
            a_ss = split[:, 2 * F:]
            s = s + a_sv * (uv0 * vv0 + uv1 * vv1 + uv2 * vv2) + a_ss
            v0 = v0 + uv0 * a_vv
            v1 = v1 + uv1 * a_vv
            v2 = v2 + uv2 * a_vv

        mu_ref[b] = (_dot(jnp.tanh(_dot(s, Wmu1_r[...]) + jnp.reshape(bmu1_r[...], (1, F))), Wmu2_r[...])
                     + jnp.reshape(bmu2_r[...], (1, F)))
        logvar = (_dot(jnp.tanh(_dot(s, Wsg1_r[...]) + jnp.reshape(bsg1_r[...], (1, F))), Wsg2_r[...])
                  + jnp.reshape(bsg2_r[...], (1, F)))
        sig_ref[b] = 1e-9 + jnp.exp(logvar * 0.5)


@jax.jit
def kernel(H, cg_adj, cg_xyz, params):
    B, N, F = H.shape
    w_args = []
    for p in params['convs']:
        w_args += [
            p['Wm1'], p['bm1'], p['Wm2'], p['bm2'], p['Wd'], p['bd'],
            p['U'], p['V'], p['Wu1'], p['bu1'], p['Wu2'], p['bu2'],
        ]
    pm, ps = params['mu'], params['sigma']
    w_args += [
        pm['W1'], pm['b1'], pm['W2'], pm['b2'],
        ps['W1'], ps['b1'], ps['W2'], ps['b2'],
    ]

    out_shape = [
        jax.ShapeDtypeStruct((B, N, F), H.dtype),
        jax.ShapeDtypeStruct((B, N, F), H.dtype),
    ]

    H_mu, H_sigma = pl.pallas_call(
        _fused_kernel,
        out_shape=out_shape,
    )(H, cg_adj, cg_xyz, *w_args)
    return H_mu, H_sigma


# speedup vs baseline: 2.0819x; 1.0396x over previous
import jax
import jax.numpy as jnp
from jax.experimental import pallas as pl

EPS = 0.001
F_DIM = 128
N_RBF = 16
CUTOFF = 5.0
NUM_CONV = 2
B_SZ = 4
N_NODES = 128

_F32 = jnp.float32


def _swish(x):
    return x * jax.nn.sigmoid(x)


def _dot(a, b):
    return jax.lax.dot_general(
        a, b, (((1,), (0,)), ((), ())), preferred_element_type=_F32)


def _fused_kernel(H_ref, adj_ref, xyz_ref, *refs):
    F = F_DIM
    conv_refs = []
    idx = 0
    for _ in range(NUM_CONV):
        conv_refs.append(refs[idx:idx + 12])
        idx += 12
    (Wmu1_r, bmu1_r, Wmu2_r, bmu2_r,
     Wsg1_r, bsg1_r, Wsg2_r, bsg2_r) = refs[idx:idx + 8]
    idx += 8
    mu_ref, sig_ref = refs[idx], refs[idx + 1]

    for b in range(B_SZ):
        s = H_ref[b]
        adj = adj_ref[b]
        xyz = xyz_ref[b]

        deg_i = jnp.sum(adj, axis=1, keepdims=True)
        dis_i = jnp.sqrt(1.0 / deg_i + EPS)
        dis_j = jnp.transpose(dis_i)

        xi, yi, zi = xyz[:, 0:1], xyz[:, 1:2], xyz[:, 2:3]
        xj = jnp.transpose(xi)
        yj = jnp.transpose(yi)
        zj = jnp.transpose(zi)
        rx = xj - xi
        ry = yj - yi
        rz = zj - zi
        dist2 = rx * rx + ry * ry + rz * rz + 1e-9
        inv_dist = jax.lax.rsqrt(dist2)
        dist = dist2 * inv_dist
        ux = rx * inv_dist
        uy = ry * inv_dist
        uz = rz * inv_dist

        t = (jnp.pi / CUTOFF) * dist
        c1 = jnp.cos(t)
        s1 = jnp.sin(t)
        env = jnp.where(dist <= CUTOFF, 0.5 * (c1 + 1.0), 0.0)

        mask = (adj > 0.0).astype(_F32)
        eew = dis_i * dis_j * mask * env
        eewd = eew * inv_dist

        sins = [s1]
        for _ in range(N_RBF - 1):
            prev2 = sins[-2] if len(sins) > 1 else jnp.zeros_like(s1)
            sins.append(2.0 * c1 * sins[-1] - prev2)
        G = [eewd * sk for sk in sins]
        G.append(eew)
        Ghat = jnp.concatenate(G, axis=1)
        eewd2 = eewd * inv_dist
        Gd = [eewd2 * sk for sk in sins]
        Gd.append(eewd)
        Gdhat = jnp.concatenate(Gd, axis=1)

        v0 = v1 = v2 = None
        for c in range(NUM_CONV):
            (Wm1_r, bm1_r, Wm2_r, bm2_r, Wd_r, bd_r,
             U_r, V_r, Wu1_r, bu1_r, Wu2_r, bu2_r) = conv_refs[c]

            bm1 = jnp.reshape(bm1_r[...], (1, F))
            bm2 = jnp.reshape(bm2_r[...], (1, 3 * F))
            phi = _dot(_swish(_dot(s, Wm1_r[...]) + bm1), Wm2_r[...]) + bm2
            phi0 = phi[:, :F]
            phi1 = phi[:, F:2 * F]
            phi2 = phi[:, 2 * F:]
            C = jnp.concatenate(
                [Wd_r[...], jnp.reshape(bd_r[...], (1, 3 * F))], axis=0)

            P = jnp.concatenate(
                [phi2, xi * phi2, yi * phi2, zi * phi2], axis=1)
            C2rep = jnp.concatenate(
                [C[:, 2 * F:]] * 4, axis=1)
            PhiA = jnp.concatenate(
                [P * C2rep[k:k + 1, :] for k in range(N_RBF + 1)], axis=0)
            SA = _dot(Gdhat, PhiA)
            S0 = SA[:, :F]
            dA0 = SA[:, F:2 * F] - xi * S0
            dA1 = SA[:, 2 * F:3 * F] - yi * S0
            dA2 = SA[:, 3 * F:] - zi * S0
            if c == 0:
                Phi1 = jnp.concatenate(
                    [phi1 * C[k:k + 1, F:2 * F] for k in range(N_RBF + 1)], axis=0)
                s = s + _dot(Ghat, Phi1)
                v0, v1, v2 = dA0, dA1, dA2
            else:
                R = jnp.concatenate([phi1, phi0 * v0, phi0 * v1, phi0 * v2], axis=1)
                Crep = jnp.concatenate(
                    [C[:, F:2 * F], C[:, :F], C[:, :F], C[:, :F]], axis=1)
                PhiR = jnp.concatenate(
                    [R * Crep[k:k + 1, :] for k in range(N_RBF + 1)], axis=0)
                M = _dot(Ghat, PhiR)
                s = s + M[:, :F]
                v0 = v0 + M[:, F:2 * F] + dA0
                v1 = v1 + M[:, 2 * F:3 * F] + dA1
                v2 = v2 + M[:, 3 * F:] + dA2

            U = U_r[...]
            V = V_r[...]
            uv0, uv1, uv2 = _dot(v0, U), _dot(v1, U), _dot(v2, U)
            vv0, vv1, vv2 = _dot(v0, V), _dot(v1, V), _dot(v2, V)
            vnorm = jnp.sqrt(vv0 * vv0 + vv1 * vv1 + vv2 * vv2 + 1e-8)
            stack = jnp.concatenate([s, vnorm], axis=1)
            inner = _swish(_dot(stack, Wu1_r[...]) + jnp.reshape(bu1_r[...], (1, F)))
            split = _dot(inner, Wu2_r[...]) + jnp.reshape(bu2_r[...], (1, 3 * F))
            a_vv = split[:, :F]
            a_sv = split[:, F:2 * F]
            a_ss = split[:, 2 * F:]
            s = s + a_sv * (uv0 * vv0 + uv1 * vv1 + uv2 * vv2) + a_ss
            v0 = v0 + uv0 * a_vv
            v1 = v1 + uv1 * a_vv
            v2 = v2 + uv2 * a_vv

        mu_ref[b] = (_dot(jnp.tanh(_dot(s, Wmu1_r[...]) + jnp.reshape(bmu1_r[...], (1, F))), Wmu2_r[...])
                     + jnp.reshape(bmu2_r[...], (1, F)))
        logvar = (_dot(jnp.tanh(_dot(s, Wsg1_r[...]) + jnp.reshape(bsg1_r[...], (1, F))), Wsg2_r[...])
                  + jnp.reshape(bsg2_r[...], (1, F)))
        sig_ref[b] = 1e-9 + jnp.exp(logvar * 0.5)


@jax.jit
def kernel(H, cg_adj, cg_xyz, params):
    B, N, F = H.shape
    w_args = []
    for p in params['convs']:
        w_args += [
            p['Wm1'], p['bm1'], p['Wm2'], p['bm2'], p['Wd'], p['bd'],
            p['U'], p['V'], p['Wu1'], p['bu1'], p['Wu2'], p['bu2'],
        ]
    pm, ps = params['mu'], params['sigma']
    w_args += [
        pm['W1'], pm['b1'], pm['W2'], pm['b2'],
        ps['W1'], ps['b1'], ps['W2'], ps['b2'],
    ]

    out_shape = [
        jax.ShapeDtypeStruct((B, N, F), H.dtype),
        jax.ShapeDtypeStruct((B, N, F), H.dtype),
    ]

    H_mu, H_sigma = pl.pallas_call(
        _fused_kernel,
        out_shape=out_shape,
    )(H, cg_adj, cg_xyz, *w_args)
    return H_mu, H_sigma


# speedup vs baseline: 2.7529x; 1.3223x over previous
import jax
import jax.numpy as jnp
from jax.experimental import pallas as pl

EPS = 0.001
F_DIM = 128
N_RBF = 16
CUTOFF = 5.0
NUM_CONV = 2
B_SZ = 4
N_NODES = 128
N_RBF_1 = N_RBF + 1

_F32 = jnp.float32


def _swish(x):
    return x * jax.nn.sigmoid(x)


def _dot(a, b):
    return jax.lax.dot_general(
        a, b, (((1,), (0,)), ((), ())), preferred_element_type=_F32)


def _fused_kernel(H_ref, adj_ref, xyz_ref, *refs):
    F = F_DIM
    N = N_NODES
    BN = B_SZ * N
    conv_refs = []
    idx = 0
    for _ in range(NUM_CONV):
        conv_refs.append(refs[idx:idx + 12])
        idx += 12
    (Wmu1_r, bmu1_r, Wmu2_r, bmu2_r,
     Wsg1_r, bsg1_r, Wsg2_r, bsg2_r) = refs[idx:idx + 8]
    idx += 8
    mu_ref, sig_ref = refs[idx], refs[idx + 1]

    geo = []
    for b in range(B_SZ):
        adj = adj_ref[b]
        xyz = xyz_ref[b]

        deg_i = jnp.sum(adj, axis=1, keepdims=True)
        dis_i = jnp.sqrt(1.0 / deg_i + EPS)
        dis_j = jnp.transpose(dis_i)

        xi, yi, zi = xyz[:, 0:1], xyz[:, 1:2], xyz[:, 2:3]
        xj = jnp.transpose(xi)
        yj = jnp.transpose(yi)
        zj = jnp.transpose(zi)
        rx = xj - xi
        ry = yj - yi
        rz = zj - zi
        dist2 = rx * rx + ry * ry + rz * rz + 1e-9
        inv_dist = jax.lax.rsqrt(dist2)
        dist = dist2 * inv_dist
        ux = rx * inv_dist
        uy = ry * inv_dist
        uz = rz * inv_dist

        t = (jnp.pi / CUTOFF) * dist
        c1 = jnp.cos(t)
        s1 = jnp.sin(t)
        env = jnp.where(dist <= CUTOFF, 0.5 * (c1 + 1.0), 0.0)

        mask = (adj > 0.0).astype(_F32)
        eew = dis_i * dis_j * mask * env
        eewd = eew * inv_dist

        sins = [s1]
        for _ in range(N_RBF - 1):
            prev2 = sins[-2] if len(sins) > 1 else jnp.zeros_like(s1)
            sins.append(2.0 * c1 * sins[-1] - prev2)
        G = [eewd * sk for sk in sins]
        G.append(eew)
        Ghat = jnp.concatenate(G, axis=1)
        Guh = jnp.concatenate(
            [jnp.concatenate([g * ux for g in G], axis=1),
             jnp.concatenate([g * uy for g in G], axis=1),
             jnp.concatenate([g * uz for g in G], axis=1)], axis=0)
        geo.append((Ghat, Guh))

    S = jnp.reshape(H_ref[...], (BN, F))
    V0 = V1 = V2 = None
    for c in range(NUM_CONV):
        (Wm1_r, bm1_r, Wm2_r, bm2_r, Wd_r, bd_r,
         U_r, V_r, Wu1_r, bu1_r, Wu2_r, bu2_r) = conv_refs[c]

        bm1 = jnp.reshape(bm1_r[...], (1, F))
        bm2 = jnp.reshape(bm2_r[...], (1, 3 * F))
        phi_all = _dot(_swish(_dot(S, Wm1_r[...]) + bm1), Wm2_r[...]) + bm2
        C = jnp.concatenate(
            [Wd_r[...], jnp.reshape(bd_r[...], (1, 3 * F))], axis=0)
        Crep = jnp.concatenate(
            [C[:, F:2 * F], C[:, :F], C[:, :F], C[:, :F]], axis=1)

        ds_l, d0_l, d1_l, d2_l = [], [], [], []
        for b in range(B_SZ):
            Ghat, Guh = geo[b]
            phi0 = phi_all[b * N:(b + 1) * N, :F]
            phi1 = phi_all[b * N:(b + 1) * N, F:2 * F]
            phi2 = phi_all[b * N:(b + 1) * N, 2 * F:]
            Phi2 = jnp.concatenate(
                [phi2 * C[k:k + 1, 2 * F:] for k in range(N_RBF + 1)], axis=0)
            MA = _dot(Guh, Phi2)
            if c == 0:
                Phi1 = jnp.concatenate(
                    [phi1 * C[k:k + 1, F:2 * F] for k in range(N_RBF + 1)],
                    axis=0)
                ds_l.append(_dot(Ghat, Phi1))
                d0_l.append(MA[:N])
                d1_l.append(MA[N:2 * N])
                d2_l.append(MA[2 * N:])
            else:
                v0 = V0[b * N:(b + 1) * N]
                v1 = V1[b * N:(b + 1) * N]
                v2 = V2[b * N:(b + 1) * N]
                R = jnp.concatenate(
                    [phi1, phi0 * v0, phi0 * v1, phi0 * v2], axis=1)
                PhiR = jnp.concatenate(
                    [R * Crep[k:k + 1, :] for k in range(N_RBF + 1)], axis=0)
                M = _dot(Ghat, PhiR)
                ds_l.append(M[:, :F])
                d0_l.append(M[:, F:2 * F] + MA[:N])
                d1_l.append(M[:, 2 * F:3 * F] + MA[N:2 * N])
                d2_l.append(M[:, 3 * F:] + MA[2 * N:])
        S = S + jnp.concatenate(ds_l, axis=0)
        if c == 0:
            V0 = jnp.concatenate(d0_l, axis=0)
            V1 = jnp.concatenate(d1_l, axis=0)
            V2 = jnp.concatenate(d2_l, axis=0)
        else:
            V0 = V0 + jnp.concatenate(d0_l, axis=0)
            V1 = V1 + jnp.concatenate(d1_l, axis=0)
            V2 = V2 + jnp.concatenate(d2_l, axis=0)

        U = U_r[...]
        V = V_r[...]
        uv0, uv1, uv2 = _dot(V0, U), _dot(V1, U), _dot(V2, U)
        vv0, vv1, vv2 = _dot(V0, V), _dot(V1, V), _dot(V2, V)
        vnorm = jnp.sqrt(vv0 * vv0 + vv1 * vv1 + vv2 * vv2 + 1e-8)
        stack = jnp.concatenate([S, vnorm], axis=1)
        inner = _swish(_dot(stack, Wu1_r[...]) + jnp.reshape(bu1_r[...], (1, F)))
        split = _dot(inner, Wu2_r[...]) + jnp.reshape(bu2_r[...], (1, 3 * F))
        a_vv = split[:, :F]
        a_sv = split[:, F:2 * F]
        a_ss = split[:, 2 * F:]
        S = S + a_sv * (uv0 * vv0 + uv1 * vv1 + uv2 * vv2) + a_ss
        V0 = V0 + uv0 * a_vv
        V1 = V1 + uv1 * a_vv
        V2 = V2 + uv2 * a_vv

    mu_all = (_dot(jnp.tanh(_dot(S, Wmu1_r[...]) + jnp.reshape(bmu1_r[...], (1, F))),
                   Wmu2_r[...]) + jnp.reshape(bmu2_r[...], (1, F)))
    logvar = (_dot(jnp.tanh(_dot(S, Wsg1_r[...]) + jnp.reshape(bsg1_r[...], (1, F))),
                   Wsg2_r[...]) + jnp.reshape(bsg2_r[...], (1, F)))
    sig_all = 1e-9 + jnp.exp(logvar * 0.5)
    mu_ref[...] = jnp.reshape(mu_all, (B_SZ, N, F))
    sig_ref[...] = jnp.reshape(sig_all, (B_SZ, N, F))


@jax.jit
def kernel(H, cg_adj, cg_xyz, params):
    B, N, F = H.shape
    w_args = []
    for p in params['convs']:
        w_args += [
            p['Wm1'], p['bm1'], p['Wm2'], p['bm2'], p['Wd'], p['bd'],
            p['U'], p['V'], p['Wu1'], p['bu1'], p['Wu2'], p['bu2'],
        ]
    pm, ps = params['mu'], params['sigma']
    w_args += [
        pm['W1'], pm['b1'], pm['W2'], pm['b2'],
        ps['W1'], ps['b1'], ps['W2'], ps['b2'],
    ]

    out_shape = [
        jax.ShapeDtypeStruct((B, N, F), H.dtype),
        jax.ShapeDtypeStruct((B, N, F), H.dtype),
    ]

    H_mu, H_sigma = pl.pallas_call(
        _fused_kernel,
        out_shape=out_shape,
    )(H, cg_adj, cg_xyz, *w_args)
    return H_mu, H_sigma
